# baseline (speedup 1.0000x reference)
import functools

import jax
import jax.numpy as jnp
from jax import lax
from jax.experimental import pallas as pl
from jax.experimental.pallas import tpu as pltpu
from jax.experimental.pallas import tpu_sc as plsc

MAX_LEN = 2048
D_HALF = 64
BATCH = 4096
HIST = 200

M = BATCH * HIST * 2
DW = D_HALF // 2
NUM_WORKERS = 32
CHUNK = M // NUM_WORKERS
K = 256
NBLK = CHUNK // K
NBUF = 2

_mesh = plsc.VectorSubcoreMesh(core_axis_name="c", subcore_axis_name="s")


@functools.partial(
    pl.kernel,
    mesh=_mesh,
    out_type=jax.ShapeDtypeStruct((M, DW), jnp.uint32),
    scratch_types=[
        pltpu.VMEM((CHUNK,), jnp.int32),
        [pltpu.VMEM((K, DW), jnp.uint32) for _ in range(NBUF)],
        pltpu.MemorySpace.VMEM_SHARED((MAX_LEN * 2, DW), jnp.uint32),
        [pltpu.SemaphoreType.DMA for _ in range(NBUF)],
        [pltpu.SemaphoreType.DMA for _ in range(NBUF)],
    ],
    compiler_params=pltpu.CompilerParams(use_tc_tiling_on_sc=False),
)
def _pe_gather(deltas_hbm, table_hbm, out_hbm, idx_v, rows, shared_tab,
               gsems, wsems):
    wid = lax.axis_index("s") * 2 + lax.axis_index("c")
    base = wid * CHUNK
    pltpu.sync_copy(deltas_hbm.at[pl.ds(base, CHUNK)], idx_v)
    parity = lax.iota(jnp.int32, 16) & 1

    @pl.when(lax.axis_index("s") == 0)
    def _stage_table():
        pltpu.sync_copy(table_hbm, shared_tab)

    plsc.subcore_barrier()

    @plsc.parallel_loop(0, CHUNK, 16, unroll=8)
    def _transform(i):
        idx_v[pl.ds(i, 16)] = idx_v[pl.ds(i, 16)] * 2 + parity

    def start_gather(g, r):
        pltpu.async_copy(
            shared_tab.at[idx_v.at[pl.ds(g * K, K)]], rows[r], gsems[r])

    for r in range(NBUF):
        start_gather(r, r)

    def body(t, carry):
        for r in range(NBUF):
            g = t * NBUF + r
            pltpu.make_async_copy(
                shared_tab.at[idx_v.at[pl.ds(0, K)]], rows[r], gsems[r]).wait()
            pltpu.async_copy(
                rows[r], out_hbm.at[pl.ds(base + g * K, K)], wsems[r])
        for r in range(NBUF):
            g_next = t * NBUF + r + NBUF
            pltpu.make_async_copy(
                rows[r], out_hbm.at[pl.ds(base, K)], wsems[r]).wait()

            @pl.when(g_next < NBLK)
            def _():
                start_gather(g_next, r)

        return carry

    lax.fori_loop(0, NBLK // NBUF, body, 0)


def kernel(deltas, pe):
    deltas_flat = deltas.reshape(M)
    table_bf = pe.astype(jnp.bfloat16).reshape(MAX_LEN * 2, DW, 2)
    table_u32 = lax.bitcast_convert_type(table_bf, jnp.uint32)
    out_u32 = _pe_gather(deltas_flat, table_u32)
    out_bf = lax.bitcast_convert_type(out_u32, jnp.bfloat16)
    return out_bf.reshape(BATCH, HIST, 2 * D_HALF).astype(jnp.float32)


# speedup vs baseline: 14.9532x; 14.9532x over previous
import functools

import jax
import jax.numpy as jnp
from jax import lax
from jax.experimental import pallas as pl
from jax.experimental.pallas import tpu as pltpu
from jax.experimental.pallas import tpu_sc as plsc

MAX_LEN = 2048
D_HALF = 64
BATCH = 4096
HIST = 200

M = BATCH * HIST * 2
DW = D_HALF // 2
NUM_WORKERS = 32
CHUNK = M // NUM_WORKERS
K = 256
NBLK = CHUNK // K
NBUF = 2

_mesh = plsc.VectorSubcoreMesh(core_axis_name="c", subcore_axis_name="s")


@functools.partial(
    pl.kernel,
    mesh=_mesh,
    out_type=jax.ShapeDtypeStruct((M, DW), jnp.uint32),
    scratch_types=[
        pltpu.VMEM((CHUNK,), jnp.int32),
        [pltpu.VMEM((K, DW), jnp.uint32) for _ in range(NBUF)],
        pltpu.MemorySpace.VMEM_SHARED((MAX_LEN * 2, DW), jnp.uint32),
        [pltpu.SemaphoreType.DMA for _ in range(NBUF)],
        [pltpu.SemaphoreType.DMA for _ in range(NBUF)],
    ],
    compiler_params=pltpu.CompilerParams(use_tc_tiling_on_sc=False),
)
def _pe_gather(deltas_hbm, table_hbm, out_hbm, idx_v, rows, shared_tab,
               gsems, wsems):
    wid = lax.axis_index("s") * 2 + lax.axis_index("c")
    base = wid * CHUNK
    pltpu.sync_copy(deltas_hbm.at[pl.ds(base, CHUNK)], idx_v)
    parity = lax.iota(jnp.int32, 16) & 1

    @pl.when(lax.axis_index("s") == 0)
    def _stage_table():
        pltpu.sync_copy(table_hbm, shared_tab)

    plsc.subcore_barrier()

    @plsc.parallel_loop(0, CHUNK, 16, unroll=8)
    def _transform(i):
        idx_v[pl.ds(i, 16)] = idx_v[pl.ds(i, 16)] * 2 + parity

    def start_gather(g, r):
        pltpu.async_copy(
            shared_tab.at[idx_v.at[pl.ds(g * K, K)]], rows[r], gsems[r])

    for r in range(NBUF):
        start_gather(r, r)

    def body(t, carry):
        for r in range(NBUF):
            g = t * NBUF + r
            pltpu.make_async_copy(
                shared_tab.at[idx_v.at[pl.ds(0, K)]], rows[r], gsems[r]).wait()
            pltpu.async_copy(
                rows[r], out_hbm.at[pl.ds(base + g * K, K)], wsems[r])
        for r in range(NBUF):
            g_next = t * NBUF + r + NBUF
            pltpu.make_async_copy(
                rows[r], out_hbm.at[pl.ds(base, K)], wsems[r]).wait()

            @pl.when(g_next < NBLK)
            def _():
                start_gather(g_next, r)

        return carry

    lax.fori_loop(0, NBLK // NBUF, body, 0)


RB = 1024


def _upcast_body(x_ref, o_ref):
    x = x_ref[...]
    lo = lax.bitcast_convert_type(x << 16, jnp.float32)
    hi = lax.bitcast_convert_type(x & jnp.uint32(0xFFFF0000), jnp.float32)
    o_ref[...] = jnp.concatenate(
        [lo[:, :DW], hi[:, :DW], lo[:, DW:], hi[:, DW:]], axis=-1)


def _upcast(out_u32):
    return pl.pallas_call(
        _upcast_body,
        grid=(M // 2 // RB,),
        in_specs=[pl.BlockSpec((RB, 2 * DW), lambda i: (i, 0))],
        out_specs=pl.BlockSpec((RB, 4 * DW), lambda i: (i, 0)),
        out_shape=jax.ShapeDtypeStruct((M // 2, 4 * DW), jnp.float32),
    )(out_u32)


def kernel(deltas, pe):
    deltas_flat = deltas.reshape(M)
    table_bf = pe.astype(jnp.bfloat16).reshape(MAX_LEN * 2, D_HALF)
    pair = jnp.stack([table_bf[:, :DW], table_bf[:, DW:]], axis=-1)
    table_u32 = lax.bitcast_convert_type(pair, jnp.uint32)
    out_u32 = _pe_gather(deltas_flat, table_u32)
    out = _upcast(out_u32.reshape(M // 2, 2 * DW))
    return out.reshape(BATCH, HIST, 2 * D_HALF)


# speedup vs baseline: 144.4830x; 9.6623x over previous
import functools

import jax
import jax.numpy as jnp
from jax import lax
from jax.experimental import pallas as pl
from jax.experimental.pallas import tpu as pltpu
from jax.experimental.pallas import tpu_sc as plsc

MAX_LEN = 2048
D_HALF = 64
BATCH = 4096
HIST = 200

PAIRS = BATCH * HIST
M = PAIRS * 2
NUM_WORKERS = 32
PCHUNK = PAIRS // NUM_WORKERS
KP = 200
NBLK = PCHUNK // KP
NBUF = 2
SK = 2048
NSTG = 2 * PCHUNK // SK

_mesh = plsc.VectorSubcoreMesh(core_axis_name="c", subcore_axis_name="s")


@functools.partial(
    pl.kernel,
    mesh=_mesh,
    out_type=jax.ShapeDtypeStruct((PAIRS, 2 * D_HALF), jnp.float32),
    scratch_types=[
        pltpu.VMEM((PCHUNK,), jnp.int32),
        pltpu.VMEM((PCHUNK,), jnp.int32),
        [pltpu.VMEM((KP, D_HALF), jnp.float32) for _ in range(NBUF)],
        [pltpu.VMEM((KP, D_HALF), jnp.float32) for _ in range(NBUF)],
        pltpu.MemorySpace.VMEM_SHARED((MAX_LEN * 2, D_HALF), jnp.float32),
        [pltpu.SemaphoreType.DMA for _ in range(NBUF)],
        [pltpu.SemaphoreType.DMA for _ in range(NBUF)],
        [pltpu.SemaphoreType.DMA for _ in range(NBUF)],
        [pltpu.SemaphoreType.DMA for _ in range(NBUF)],
    ],
    compiler_params=pltpu.CompilerParams(use_tc_tiling_on_sc=False, needs_layout_passes=False),
)
def _pe_gather(idx0_hbm, idx1_hbm, table_hbm, out_hbm, idx0_v, idx1_v,
               rows_e, rows_o, shared_tab, gesems, gosems,
               wesems, wosems):
    wid = lax.axis_index("s") * 2 + lax.axis_index("c")
    pbase = wid * PCHUNK

    @pl.when(lax.axis_index("s") == 0)
    def _stage_table():
        pltpu.sync_copy(table_hbm, shared_tab)

    pltpu.sync_copy(idx0_hbm.at[pl.ds(pbase, PCHUNK)], idx0_v)
    pltpu.sync_copy(idx1_hbm.at[pl.ds(pbase, PCHUNK)], idx1_v)
    plsc.subcore_barrier()

    def start_gather(g, r):
        pltpu.async_copy(
            shared_tab.at[idx0_v.at[pl.ds(g * KP, KP)]], rows_e[r],
            gesems[r])
        pltpu.async_copy(
            shared_tab.at[idx1_v.at[pl.ds(g * KP, KP)]], rows_o[r],
            gosems[r])

    for r in range(NBUF):
        start_gather(r, r)

    def body(t, carry):
        for r in range(NBUF):
            g = t * NBUF + r
            off = pbase + g * KP
            pltpu.make_async_copy(
                shared_tab.at[idx0_v.at[pl.ds(0, KP)]], rows_e[r],
                gesems[r]).wait()
            pltpu.async_copy(
                rows_e[r],
                out_hbm.at[pl.ds(off, KP), pl.ds(0, D_HALF)], wesems[r])
            pltpu.make_async_copy(
                shared_tab.at[idx1_v.at[pl.ds(0, KP)]], rows_o[r],
                gosems[r]).wait()
            pltpu.async_copy(
                rows_o[r],
                out_hbm.at[pl.ds(off, KP), pl.ds(D_HALF, D_HALF)],
                wosems[r])
        for r in range(NBUF):
            g_next = t * NBUF + r + NBUF
            pltpu.make_async_copy(
                rows_e[r],
                out_hbm.at[pl.ds(pbase, KP), pl.ds(0, D_HALF)],
                wesems[r]).wait()
            pltpu.make_async_copy(
                rows_o[r],
                out_hbm.at[pl.ds(pbase, KP), pl.ds(D_HALF, D_HALF)],
                wosems[r]).wait()

            @pl.when(g_next < NBLK)
            def _():
                start_gather(g_next, r)

        return carry

    lax.fori_loop(0, NBLK // NBUF, body, 0)


def kernel(deltas, pe):
    idx0 = deltas[..., 0].reshape(PAIRS) * 2
    idx1 = deltas[..., 1].reshape(PAIRS) * 2 + 1
    table = pe.reshape(MAX_LEN * 2, D_HALF)
    out = _pe_gather(idx0, idx1, table)
    return out.reshape(BATCH, HIST, 2 * D_HALF)
